# baseline (speedup 1.0000x reference)
import functools

import jax
import jax.numpy as jnp
from jax import lax
from jax.experimental import pallas as pl
from jax.experimental.pallas import tpu as pltpu
from jax.experimental.pallas import tpu_sc as plsc

_B, _N, _D = 4096, 200, 64
_TOTAL = _B * _N
_NC, _NS = 2, 16
_NW = _NC * _NS
_PER_W = _TOTAL // _NW
_CHUNK = 512
_NCHUNK = _PER_W // _CHUNK


def _embed_lookup(tokens_flat, weight):
    mesh = plsc.VectorSubcoreMesh(core_axis_name="c", subcore_axis_name="s")

    @functools.partial(
        pl.kernel,
        mesh=mesh,
        compiler_params=pltpu.CompilerParams(use_tc_tiling_on_sc=False),
        out_type=jax.ShapeDtypeStruct((_TOTAL, _D), jnp.float32),
        scratch_types=[
            pltpu.VMEM((_CHUNK,), jnp.int32),
            pltpu.VMEM((_CHUNK, _D), jnp.float32),
            pltpu.SemaphoreType.DMA,
        ],
    )
    def k(idx_hbm, table_hbm, out_hbm, idx_v, rows_v, sem):
        wid = lax.axis_index("s") * _NC + lax.axis_index("c")
        base = wid * _PER_W

        def body(i, carry):
            off = base + i * _CHUNK
            pltpu.sync_copy(idx_hbm.at[pl.ds(off, _CHUNK)], idx_v)
            pltpu.async_copy(table_hbm.at[idx_v], rows_v, sem).wait()
            pltpu.sync_copy(rows_v, out_hbm.at[pl.ds(off, _CHUNK)])
            return carry

        lax.fori_loop(0, _NCHUNK, body, 0)

    return k(tokens_flat, weight)


def kernel(tokens, weight):
    flat = _embed_lookup(tokens.reshape(-1).astype(jnp.int32), weight)
    return flat.reshape(_B, _N, _D)


# speedup vs baseline: 1.0449x; 1.0449x over previous
import functools

import jax
import jax.numpy as jnp
from jax import lax
from jax.experimental import pallas as pl
from jax.experimental.pallas import tpu as pltpu
from jax.experimental.pallas import tpu_sc as plsc

_B, _N, _D = 4096, 200, 64
_TOTAL = _B * _N
_NC, _NS = 2, 16
_NW = _NC * _NS
_PER_W = _TOTAL // _NW
_CHUNK = 512
_NCHUNK = _PER_W // _CHUNK
_NBUF = 3


def _embed_lookup(tokens_flat, weight):
    mesh = plsc.VectorSubcoreMesh(core_axis_name="c", subcore_axis_name="s")

    @functools.partial(
        pl.kernel,
        mesh=mesh,
        compiler_params=pltpu.CompilerParams(use_tc_tiling_on_sc=False),
        out_type=jax.ShapeDtypeStruct((_TOTAL, _D), jnp.float32),
        scratch_types=[
            pltpu.VMEM((_PER_W,), jnp.int32),
            pltpu.VMEM((_CHUNK, _D), jnp.float32),
            pltpu.VMEM((_CHUNK, _D), jnp.float32),
            pltpu.VMEM((_CHUNK, _D), jnp.float32),
            pltpu.SemaphoreType.DMA,
            pltpu.SemaphoreType.DMA,
            pltpu.SemaphoreType.DMA,
            pltpu.SemaphoreType.DMA,
            pltpu.SemaphoreType.DMA,
            pltpu.SemaphoreType.DMA,
        ],
    )
    def k(idx_hbm, table_hbm, out_hbm, idx_v,
          rows0, rows1, rows2, g0, g1, g2, s0, s1, s2):
        rows = [rows0, rows1, rows2]
        gsem = [g0, g1, g2]
        ssem = [s0, s1, s2]
        wid = lax.axis_index("s") * _NC + lax.axis_index("c")
        base = wid * _PER_W

        pltpu.sync_copy(idx_hbm.at[pl.ds(base, _PER_W)], idx_v)

        def start_g(g, b):
            pltpu.async_copy(
                table_hbm.at[idx_v.at[pl.ds(g * _CHUNK, _CHUNK)]],
                rows[b], gsem[b])

        def wait_g(b):
            pltpu.make_async_copy(
                table_hbm.at[idx_v.at[pl.ds(0, _CHUNK)]],
                rows[b], gsem[b]).wait()

        def start_s(g, b):
            pltpu.async_copy(
                rows[b], out_hbm.at[pl.ds(base + g * _CHUNK, _CHUNK)],
                ssem[b])

        def wait_s(b):
            pltpu.make_async_copy(
                rows[b], out_hbm.at[pl.ds(base, _CHUNK)], ssem[b]).wait()

        start_g(0, 0)
        start_g(1, 1)

        wait_g(0)
        start_s(0, 0)
        start_g(2, 2)

        wait_g(1)
        start_s(1, 1)
        wait_s(0)
        start_g(3, 0)

        wait_g(2)
        start_s(2, 2)
        wait_s(1)
        start_g(4, 1)

        def body(blk, carry):
            base_g = blk * _NBUF
            for j in range(_NBUF):
                g = base_g + j
                wait_g(j)
                start_s(g, j)
                bn = (j + _NBUF - 1) % _NBUF
                wait_s(bn)
                start_g(g + _NBUF - 1, bn)
            return carry

        lax.fori_loop(1, _NCHUNK // _NBUF - 1, body, 0)

        g_tail = (_NCHUNK // _NBUF - 1) * _NBUF
        for t in range(_NCHUNK - g_tail):
            g = g_tail + t
            b = g % _NBUF
            wait_g(b)
            start_s(g, b)
            ng = g + _NBUF - 1
            if ng < _NCHUNK:
                bn = (b + _NBUF - 1) % _NBUF
                wait_s(bn)
                start_g(ng, bn)

        for b in range(_NBUF):
            wait_s(b)

    return k(tokens_flat, weight)


def kernel(tokens, weight):
    flat = _embed_lookup(tokens.reshape(-1).astype(jnp.int32), weight)
    return flat.reshape(_B, _N, _D)
